# baseline (speedup 1.0000x reference)
import functools

import jax
import jax.numpy as jnp
from jax import lax
from jax.experimental import pallas as pl
from jax.experimental.pallas import tpu as pltpu
from jax.experimental.pallas import tpu_sc as plsc

HIDDEN = 64
SEQ = 2048
BATCH = 4
TOTAL = BATCH * SEQ
NC, NS = 2, 16
NW = NC * NS
B_PER_W = TOTAL // NW
CHUNK = 128
N_CHUNKS = B_PER_W // CHUNK


def _make_kernel():
    mesh = plsc.VectorSubcoreMesh(core_axis_name="c", subcore_axis_name="s")

    @functools.partial(
        pl.kernel,
        mesh=mesh,
        compiler_params=pltpu.CompilerParams(use_tc_tiling_on_sc=False),
        out_type=jax.ShapeDtypeStruct((TOTAL, HIDDEN), jnp.float32),
        scratch_types=[
            pltpu.VMEM((N_CHUNKS, CHUNK), jnp.int32),
            pltpu.VMEM((B_PER_W, HIDDEN), jnp.float32),
            pltpu.VMEM((B_PER_W, HIDDEN), jnp.float32),
            pltpu.SemaphoreType.DMA,
        ],
    )
    def body(x_hbm, emb_hbm, pos_hbm, out_hbm, idx_v, rows_v, pos_v, sem):
        wid = lax.axis_index("s") * NC + lax.axis_index("c")
        base = wid * B_PER_W
        pos_base = lax.rem(base, SEQ)

        pltpu.sync_copy(x_hbm.at[pl.ds(wid * N_CHUNKS, N_CHUNKS)], idx_v)
        copies = [
            pltpu.async_copy(
                emb_hbm.at[idx_v.at[j]],
                rows_v.at[pl.ds(j * CHUNK, CHUNK)],
                sem,
            )
            for j in range(N_CHUNKS)
        ]
        pltpu.sync_copy(pos_hbm.at[pl.ds(pos_base, B_PER_W)], pos_v)
        for cp in copies:
            cp.wait()

        scale = jnp.float32(8.0)

        def step(i, carry):
            for j in range(HIDDEN // 16):
                sl = pl.ds(j * 16, 16)
                rows_v[i, sl] = rows_v[i, sl] * scale + pos_v[i, sl]
            return carry

        lax.fori_loop(0, B_PER_W, step, 0)

        pltpu.sync_copy(rows_v, out_hbm.at[pl.ds(base, B_PER_W)])

    return body


def kernel(x, emb_table, pos_table):
    xf = x.reshape(NW * N_CHUNKS, CHUNK).astype(jnp.int32)
    out = _make_kernel()(xf, emb_table, pos_table)
    return out.reshape(BATCH, SEQ, HIDDEN)
